# baseline (speedup 1.0000x reference)
import jax
import jax.numpy as jnp
from jax.experimental import pallas as pl
from jax.experimental.pallas import tpu as pltpu

_NE = 1024
_D = 64
_R = 512
_NT = 16 * 32 * 32
_G = _NT // _R


def _vq_body(xt_ref, xlin_ref, cbt_ref, cb_ref, a2_ref, b2_ref,
             idx_ref, qst_ref, loss_ref):
    g = pl.program_id(0)
    xb = xt_ref[...]
    cb = cb_ref[...]

    a2 = a2_ref[...]
    b2 = b2_ref[...]
    ab = jnp.dot(xb, cbt_ref[...],
                 preferred_element_type=jnp.float32)
    d2 = a2 + b2 - 2.0 * ab
    dist = jnp.sqrt(jnp.maximum(d2, 0.0))
    m = jnp.min(dist, axis=1, keepdims=True)
    lanes = jax.lax.broadcasted_iota(jnp.int32, dist.shape, 1)
    idx = jnp.min(jnp.where(dist == m, lanes, jnp.int32(_NE)), axis=1)
    idx_ref[0, 0, :] = idx

    enc = (idx[:, None] == jax.lax.broadcasted_iota(jnp.int32, (1, _NE), 1)
           ).astype(jnp.float32)
    q = jnp.dot(enc, cb, preferred_element_type=jnp.float32)

    xl = xlin_ref[...]
    diff = q - xl
    qst_ref[...] = xl + diff

    @pl.when(g == 0)
    def _init():
        loss_ref[0, 0] = 0.0
    loss_ref[0, 0] += jnp.sum(diff * diff)


def kernel(x, codebook):
    B, C, H, W = x.shape
    xt = jnp.transpose(x, (0, 2, 3, 1)).reshape(_NT, _D)
    xlin = x.reshape(_NT, _D)
    cbt = codebook.T
    a2 = jnp.sum(xt * xt, axis=1, keepdims=True)
    b2 = jnp.sum(codebook * codebook, axis=1)[None, :]

    idx3, qst2, loss_acc = pl.pallas_call(
        _vq_body,
        grid=(_G,),
        in_specs=[
            pl.BlockSpec((_R, _D), lambda g: (g, 0)),
            pl.BlockSpec((_R, _D), lambda g: (g, 0)),
            pl.BlockSpec((_D, _NE), lambda g: (0, 0)),
            pl.BlockSpec((_NE, _D), lambda g: (0, 0)),
            pl.BlockSpec((_R, 1), lambda g: (g, 0)),
            pl.BlockSpec((1, _NE), lambda g: (0, 0)),
        ],
        out_specs=[
            pl.BlockSpec((1, 1, _R), lambda g: (g, 0, 0)),
            pl.BlockSpec((_R, _D), lambda g: (g, 0)),
            pl.BlockSpec(memory_space=pltpu.SMEM, block_shape=(1, 1),
                         index_map=lambda g: (0, 0)),
        ],
        out_shape=[
            jax.ShapeDtypeStruct((_G, 1, _R), jnp.int32),
            jax.ShapeDtypeStruct((_NT, _D), jnp.float32),
            jax.ShapeDtypeStruct((1, 1), jnp.float32),
        ],
    )(xt, xlin, cbt, codebook, a2, b2)

    quantized_st = qst2.reshape(B, C, H, W)
    m = loss_acc[0, 0] / jnp.float32(_NT * _D)
    loss = m * jnp.float32(0.25) + m
    indices = idx3.reshape(B, H, W)
    return quantized_st, loss, indices


# speedup vs baseline: 1.0689x; 1.0689x over previous
import jax
import jax.numpy as jnp
from jax.experimental import pallas as pl
from jax.experimental.pallas import tpu as pltpu

_NE = 1024
_D = 64
_R = 1024
_NT = 16 * 32 * 32
_G = _NT // _R


def _vq_body(x3_ref, xlin_ref, cb_ref, cbt_ref, a2_ref, b2_ref,
             idx_ref, qst_ref, loss_ref):
    g = pl.program_id(0)
    xbT = x3_ref[0]
    cb = cb_ref[...]

    abT = jax.lax.dot_general(cb, xbT, (((1,), (0,)), ((), ())),
                              preferred_element_type=jnp.float32)
    d2 = (a2_ref[...] + b2_ref[...]) - 2.0 * abT
    dist = jnp.sqrt(jnp.maximum(d2, 0.0))
    m = jnp.min(dist, axis=0, keepdims=True)
    rows = jax.lax.broadcasted_iota(jnp.int32, dist.shape, 0)
    idx = jnp.min(jnp.where(dist == m, rows, jnp.int32(_NE)), axis=0)
    idx_ref[0, 0, :] = idx

    encT = (jax.lax.broadcasted_iota(jnp.int32, (_NE, _R), 0) == idx[None, :]
            ).astype(jnp.float32)
    q = jax.lax.dot_general(encT, cb, (((0,), (0,)), ((), ())),
                            preferred_element_type=jnp.float32)

    xl = xlin_ref[...]
    diff = q - xl
    qst_ref[...] = xl + diff

    @pl.when(g == 0)
    def _init():
        loss_ref[0, 0] = 0.0
    loss_ref[0, 0] += jnp.sum(diff * diff)


def kernel(x, codebook):
    B, C, H, W = x.shape
    x3 = x.reshape(_G, _D, _R)
    xlin = x.reshape(_NT, _D)
    cbt = codebook.T
    xt = jnp.transpose(x, (0, 2, 3, 1)).reshape(_NT, _D)
    a2 = jnp.sum(xt * xt, axis=1)[None, :]
    b2 = jnp.sum(codebook * codebook, axis=1)[:, None]

    idx3, qst2, loss_acc = pl.pallas_call(
        _vq_body,
        grid=(_G,),
        in_specs=[
            pl.BlockSpec((1, _D, _R), lambda g: (g, 0, 0)),
            pl.BlockSpec((_R, _D), lambda g: (g, 0)),
            pl.BlockSpec((_NE, _D), lambda g: (0, 0)),
            pl.BlockSpec((_D, _NE), lambda g: (0, 0)),
            pl.BlockSpec((1, _R), lambda g: (0, g)),
            pl.BlockSpec((_NE, 1), lambda g: (0, 0)),
        ],
        out_specs=[
            pl.BlockSpec((1, 1, _R), lambda g: (g, 0, 0)),
            pl.BlockSpec((_R, _D), lambda g: (g, 0)),
            pl.BlockSpec(memory_space=pltpu.SMEM, block_shape=(1, 1),
                         index_map=lambda g: (0, 0)),
        ],
        out_shape=[
            jax.ShapeDtypeStruct((_G, 1, _R), jnp.int32),
            jax.ShapeDtypeStruct((_NT, _D), jnp.float32),
            jax.ShapeDtypeStruct((1, 1), jnp.float32),
        ],
    )(x3, xlin, codebook, cbt, a2, b2)

    quantized_st = qst2.reshape(B, C, H, W)
    m = loss_acc[0, 0] / jnp.float32(_NT * _D)
    loss = m * jnp.float32(0.25) + m
    indices = idx3.reshape(B, H, W)
    return quantized_st, loss, indices


# speedup vs baseline: 1.0820x; 1.0123x over previous
import jax
import jax.numpy as jnp
from jax.experimental import pallas as pl
from jax.experimental.pallas import tpu as pltpu

_NE = 1024
_D = 64
_R = 1024
_NT = 16 * 32 * 32
_G = _NT // _R


def _tr_body(x3_ref, xt_ref):
    xt_ref[...] = jnp.transpose(x3_ref[0], (1, 0))


def _vq_body(x3_ref, xlin_ref, cb_ref, a2_ref, b2_ref,
             idx_ref, qst_ref, loss_ref):
    g = pl.program_id(0)
    xbT = x3_ref[0]
    cb = cb_ref[...]

    abT = jax.lax.dot_general(cb, xbT, (((1,), (0,)), ((), ())),
                              preferred_element_type=jnp.float32)
    d2 = (a2_ref[...] + b2_ref[...]) - 2.0 * abT
    dist = jnp.sqrt(jnp.maximum(d2, 0.0))
    m = jnp.min(dist, axis=0, keepdims=True)
    rowsf = jax.lax.broadcasted_iota(jnp.int32, dist.shape, 0
                                     ).astype(jnp.float32)
    cnd = jnp.where(dist == m, rowsf, jnp.float32(_NE))
    idxf = jnp.min(cnd, axis=0)
    idx_ref[0, 0, :] = idxf.astype(jnp.int32)

    encT = (cnd == idxf[None, :]).astype(jnp.float32)
    q = jax.lax.dot_general(encT, cb, (((0,), (0,)), ((), ())),
                            preferred_element_type=jnp.float32)

    xl = xlin_ref[...]
    diff = q - xl
    qst_ref[...] = xl + diff

    @pl.when(g == 0)
    def _init():
        loss_ref[0, 0] = 0.0
    loss_ref[0, 0] += jnp.sum(diff * diff)


def kernel(x, codebook):
    B, C, H, W = x.shape
    x3 = x.reshape(_G, _D, _R)
    xlin = x.reshape(_NT, _D)

    xt = jnp.transpose(x, (0, 2, 3, 1)).reshape(_NT, _D)
    a2 = jnp.sum(xt * xt, axis=1)[None, :]
    b2 = jnp.sum(codebook * codebook, axis=1)[:, None]

    idx3, qst2, loss_acc = pl.pallas_call(
        _vq_body,
        grid=(_G,),
        in_specs=[
            pl.BlockSpec((1, _D, _R), lambda g: (g, 0, 0)),
            pl.BlockSpec((_R, _D), lambda g: (g, 0)),
            pl.BlockSpec((_NE, _D), lambda g: (0, 0)),
            pl.BlockSpec((1, _R), lambda g: (0, g)),
            pl.BlockSpec((_NE, 1), lambda g: (0, 0)),
        ],
        out_specs=[
            pl.BlockSpec((1, 1, _R), lambda g: (g, 0, 0)),
            pl.BlockSpec((_R, _D), lambda g: (g, 0)),
            pl.BlockSpec(memory_space=pltpu.SMEM, block_shape=(1, 1),
                         index_map=lambda g: (0, 0)),
        ],
        out_shape=[
            jax.ShapeDtypeStruct((_G, 1, _R), jnp.int32),
            jax.ShapeDtypeStruct((_NT, _D), jnp.float32),
            jax.ShapeDtypeStruct((1, 1), jnp.float32),
        ],
    )(x3, xlin, codebook, a2, b2)

    quantized_st = qst2.reshape(B, C, H, W)
    m = loss_acc[0, 0] / jnp.float32(_NT * _D)
    loss = m * jnp.float32(0.25) + m
    indices = idx3.reshape(B, H, W)
    return quantized_st, loss, indices
